# baseline (speedup 1.0000x reference)
import functools

import jax
import jax.numpy as jnp
from jax import lax
from jax.experimental import pallas as pl
from jax.experimental.pallas import tpu as pltpu
from jax.experimental.pallas import tpu_sc as plsc

NSEG = 1000
SPAD = 1024
PAD_ID = 1016
NC, NS = 2, 16
NW = NC * NS
CH = 128
LANES = 16
DPAD = 256


def _k1(x_ref, brow_ref, w1_ref, b1_ref, w2_ref, b2_ref, rw1_ref,
        out1_ref, s_ref):
    i = pl.program_id(0)
    xb = x_ref[...]
    h = jnp.maximum(
        jnp.dot(xb, w1_ref[...], preferred_element_type=jnp.float32)
        + b1_ref[...], 0.0)
    out1_ref[...] = (
        jnp.dot(h, w2_ref[...], preferred_element_type=jnp.float32)
        + b2_ref[...])
    y = jnp.dot(xb, rw1_ref[...], preferred_element_type=jnp.float32)
    b = brow_ref[0]
    nb = b.shape[1]
    seg = jax.lax.broadcasted_iota(jnp.int32, (SPAD, nb), 0).astype(jnp.float32)
    oh_t = (jnp.broadcast_to(b, (SPAD, nb)) == seg).astype(jnp.float32)
    part = jnp.dot(oh_t, y, preferred_element_type=jnp.float32)

    @pl.when(i == 0)
    def _():
        s_ref[...] = part

    @pl.when(i > 0)
    def _():
        s_ref[...] += part


def _krho(s_ref, rb1_ref, rw2_ref, rb2_ref, r_ref):
    r_ref[...] = (
        jnp.dot(jnp.maximum(s_ref[...] + rb1_ref[...], 0.0), rw2_ref[...],
                preferred_element_type=jnp.float32)
        + rb2_ref[...])


def _gadd_body(nchunk, tail, d_out, out1_hbm, r_hbm, b2d_hbm, out_hbm,
               ibuf, o1buf, rbuf, sem):
    c = lax.axis_index("c")
    sid = lax.axis_index("s")
    w = sid * NC + c
    groups = d_out // LANES
    cpw = (nchunk + NW - 1) // NW

    def body(k, carry):
        g = w + NW * k

        @pl.when(g < nchunk)
        def _():
            pltpu.sync_copy(b2d_hbm.at[g], ibuf)
            pltpu.sync_copy(out1_hbm.at[pl.ds(g * CH, CH)], o1buf)
            pltpu.async_copy(r_hbm.at[ibuf], rbuf, sem).wait()

            def arow(rr, carry2):
                for j in range(groups):
                    sl = (rr, pl.ds(j * LANES, LANES))
                    o1buf[sl] = o1buf[sl] + rbuf[sl]
                return carry2
            lax.fori_loop(0, CH, arow, 0)

            @pl.when(g < nchunk - 1)
            def _():
                pltpu.sync_copy(o1buf, out_hbm.at[pl.ds(g * CH, CH)])

            @pl.when(g == nchunk - 1)
            def _():
                pltpu.sync_copy(o1buf.at[pl.ds(0, tail)],
                                out_hbm.at[pl.ds(g * CH, tail)])
        return carry

    lax.fori_loop(0, cpw, body, 0)


def kernel(x, batch, phi_w1, phi_b1, phi_w2, phi_b2,
           rho_w1, rho_b1, rho_w2, rho_b2):
    n, d_in = x.shape
    d_hid = phi_w1.shape[1]
    d_out = phi_w2.shape[1]
    bsz = 1000
    nblk = n // bsz
    npad = ((n + CH - 1) // CH) * CH
    nchunk = npad // CH
    tail = n - (nchunk - 1) * CH

    b1 = phi_b1.reshape(1, d_hid)
    b2 = phi_b2.reshape(1, d_out)
    rb1 = rho_b1.reshape(1, d_hid)
    rw2p = jnp.pad(rho_w2, ((0, 0), (0, DPAD - d_out)))
    rb2p = jnp.pad(rho_b2.reshape(1, d_out), ((0, 0), (0, DPAD - d_out)))

    bf = batch.astype(jnp.float32)
    brow = bf.reshape(nblk, 1, bsz)
    bi = batch.astype(jnp.int32)
    b2d = jnp.concatenate(
        [bi, jnp.full((npad - n,), PAD_ID, jnp.int32)]).reshape(nchunk, CH)

    full = lambda i: (0, 0)
    out1, s = pl.pallas_call(
        _k1,
        grid=(nblk,),
        in_specs=[
            pl.BlockSpec((bsz, d_in), lambda i: (i, 0)),
            pl.BlockSpec((1, 1, bsz), lambda i: (i, 0, 0)),
            pl.BlockSpec((d_in, d_hid), full),
            pl.BlockSpec((1, d_hid), full),
            pl.BlockSpec((d_hid, d_out), full),
            pl.BlockSpec((1, d_out), full),
            pl.BlockSpec((d_in, d_hid), full),
        ],
        out_specs=[
            pl.BlockSpec((bsz, d_out), lambda i: (i, 0)),
            pl.BlockSpec((SPAD, d_hid), full),
        ],
        out_shape=[
            jax.ShapeDtypeStruct((npad, d_out), jnp.float32),
            jax.ShapeDtypeStruct((SPAD, d_hid), jnp.float32),
        ],
    )(x, brow, phi_w1, b1, phi_w2, b2, rho_w1)

    r = pl.pallas_call(
        _krho,
        grid=(1,),
        in_specs=[
            pl.BlockSpec((SPAD, d_hid), lambda i: (0, 0)),
            pl.BlockSpec((1, d_hid), full),
            pl.BlockSpec((d_hid, DPAD), full),
            pl.BlockSpec((1, DPAD), full),
        ],
        out_specs=pl.BlockSpec((SPAD, DPAD), lambda i: (0, 0)),
        out_shape=jax.ShapeDtypeStruct((SPAD, DPAD), jnp.float32),
    )(s, rb1, rw2p, rb2p)

    mesh = plsc.VectorSubcoreMesh(
        core_axis_name="c", subcore_axis_name="s",
        num_cores=NC, num_subcores=NS)

    out = pl.kernel(
        functools.partial(_gadd_body, nchunk, tail, d_out),
        out_type=jax.ShapeDtypeStruct((n, d_out), jnp.float32),
        mesh=mesh,
        scratch_types=[
            pltpu.VMEM((CH,), jnp.int32),
            pltpu.VMEM((CH, d_out), jnp.float32),
            pltpu.VMEM((CH, DPAD), jnp.float32),
            pltpu.SemaphoreType.DMA,
        ],
    )(out1, r, b2d)
    return out


# speedup vs baseline: 1.2187x; 1.2187x over previous
import functools

import jax
import jax.numpy as jnp
from jax import lax
from jax.experimental import pallas as pl
from jax.experimental.pallas import tpu as pltpu
from jax.experimental.pallas import tpu_sc as plsc

NSEG = 1000
SPAD = 1024
PAD_ID = 1016
NC, NS = 2, 16
NW = NC * NS
CH = 96
LANES = 16
DPAD = 256


def _k1(x_ref, brow_ref, w1_ref, b1_ref, w2_ref, b2_ref, rw1_ref,
        out1_ref, s_ref):
    i = pl.program_id(0)
    xb = x_ref[...]
    h = jnp.maximum(
        jnp.dot(xb, w1_ref[...], preferred_element_type=jnp.float32)
        + b1_ref[...], 0.0)
    out1_ref[...] = (
        jnp.dot(h, w2_ref[...], preferred_element_type=jnp.float32)
        + b2_ref[...])
    y = jnp.dot(xb, rw1_ref[...], preferred_element_type=jnp.float32)
    b = brow_ref[0]
    nb = b.shape[1]
    seg = jax.lax.broadcasted_iota(jnp.int32, (SPAD, nb), 0).astype(jnp.float32)
    oh_t = (jnp.broadcast_to(b, (SPAD, nb)) == seg).astype(jnp.float32)
    part = jnp.dot(oh_t, y, preferred_element_type=jnp.float32)

    @pl.when(i == 0)
    def _():
        s_ref[...] = part

    @pl.when(i > 0)
    def _():
        s_ref[...] += part


def _krho(s_ref, rb1_ref, rw2_ref, rb2_ref, r_ref):
    r_ref[...] = (
        jnp.dot(jnp.maximum(s_ref[...] + rb1_ref[...], 0.0), rw2_ref[...],
                preferred_element_type=jnp.float32)
        + rb2_ref[...])


def _gadd_body(nchunk, tail, d_out, cpw, out1_hbm, r_hbm, b3d_hbm, out_hbm,
               iball, o1a, o1b, ra, rb, sla, slb, sga, sgb, ssa, ssb):
    c = lax.axis_index("c")
    sid = lax.axis_index("s")
    w = sid * NC + c
    groups = d_out // LANES
    o1 = [o1a, o1b]
    rbf = [ra, rb]
    sld = [sla, slb]
    sg = [sga, sgb]
    sst = [ssa, ssb]
    LAST = cpw - 1
    wlim = nchunk - NW * LAST

    pltpu.sync_copy(b3d_hbm.at[w], iball)

    def guarded(k, fn):
        if k == LAST:
            @pl.when(w < wlim)
            def _():
                fn()
        else:
            fn()

    def issue_loads(k):
        b = k % 2
        pltpu.async_copy(out1_hbm.at[pl.ds((w + NW * k) * CH, CH)],
                         o1[b], sld[b])
        pltpu.async_copy(r_hbm.at[iball.at[k]], rbf[b], sg[b])

    def wait_loads(k):
        b = k % 2
        pltpu.make_async_copy(out1_hbm.at[pl.ds(0, CH)], o1[b],
                              sld[b]).wait()
        pltpu.make_async_copy(r_hbm.at[iball.at[k]], rbf[b], sg[b]).wait()

    def add_chunk(k):
        b = k % 2
        o1b_, rb_ = o1[b], rbf[b]

        def arow(q, carry):
            base = q * 4
            for t in range(4):
                for j in range(groups):
                    sl = (base + t, pl.ds(j * LANES, LANES))
                    o1b_[sl] = o1b_[sl] + rb_[sl]
            return carry
        lax.fori_loop(0, CH // 4, arow, 0)

    def issue_store(k):
        b = k % 2
        if k == LAST:
            @pl.when(w == wlim - 1)
            def _():
                pltpu.async_copy(o1[b].at[pl.ds(0, tail)],
                                 out_hbm.at[pl.ds((w + NW * k) * CH, tail)],
                                 sst[b])

            @pl.when(w < wlim - 1)
            def _():
                pltpu.async_copy(o1[b],
                                 out_hbm.at[pl.ds((w + NW * k) * CH, CH)],
                                 sst[b])
        else:
            pltpu.async_copy(o1[b], out_hbm.at[pl.ds((w + NW * k) * CH, CH)],
                             sst[b])

    def wait_store(k):
        b = k % 2
        if k == LAST:
            @pl.when(w == wlim - 1)
            def _():
                pltpu.make_async_copy(o1[b].at[pl.ds(0, tail)],
                                      out_hbm.at[pl.ds(0, tail)],
                                      sst[b]).wait()

            @pl.when(w < wlim - 1)
            def _():
                pltpu.make_async_copy(o1[b], out_hbm.at[pl.ds(0, CH)],
                                      sst[b]).wait()
        else:
            pltpu.make_async_copy(o1[b], out_hbm.at[pl.ds(0, CH)],
                                  sst[b]).wait()

    for k in range(min(2, cpw)):
        guarded(k, lambda k=k: issue_loads(k))
    for k in range(cpw):
        guarded(k, lambda k=k: wait_loads(k))
        guarded(k, lambda k=k: add_chunk(k))
        guarded(k, lambda k=k: issue_store(k))
        if k + 2 < cpw:
            wait_store(k)
            guarded(k + 2, lambda k=k: issue_loads(k + 2))
    for k in range(max(cpw - 2, 0), cpw):
        guarded(k, lambda k=k: wait_store(k))


def kernel(x, batch, phi_w1, phi_b1, phi_w2, phi_b2,
           rho_w1, rho_b1, rho_w2, rho_b2):
    n, d_in = x.shape
    d_hid = phi_w1.shape[1]
    d_out = phi_w2.shape[1]
    bsz = 1000
    nblk = n // bsz
    npad = ((n + CH - 1) // CH) * CH
    nchunk = npad // CH
    tail = n - (nchunk - 1) * CH

    b1 = phi_b1.reshape(1, d_hid)
    b2 = phi_b2.reshape(1, d_out)
    rb1 = rho_b1.reshape(1, d_hid)
    rw2p = jnp.pad(rho_w2, ((0, 0), (0, DPAD - d_out)))
    rb2p = jnp.pad(rho_b2.reshape(1, d_out), ((0, 0), (0, DPAD - d_out)))

    bf = batch.astype(jnp.float32)
    brow = bf.reshape(nblk, 1, bsz)
    bi = batch.astype(jnp.int32)
    cpw = (nchunk + NW - 1) // NW
    nck_pad = cpw * NW
    b2d = jnp.concatenate(
        [bi, jnp.full((nck_pad * CH - n,), PAD_ID, jnp.int32)])
    b3d = b2d.reshape(cpw, NW, CH).transpose(1, 0, 2)

    full = lambda i: (0, 0)
    out1, s = pl.pallas_call(
        _k1,
        grid=(nblk,),
        in_specs=[
            pl.BlockSpec((bsz, d_in), lambda i: (i, 0)),
            pl.BlockSpec((1, 1, bsz), lambda i: (i, 0, 0)),
            pl.BlockSpec((d_in, d_hid), full),
            pl.BlockSpec((1, d_hid), full),
            pl.BlockSpec((d_hid, d_out), full),
            pl.BlockSpec((1, d_out), full),
            pl.BlockSpec((d_in, d_hid), full),
        ],
        out_specs=[
            pl.BlockSpec((bsz, d_out), lambda i: (i, 0)),
            pl.BlockSpec((SPAD, d_hid), full),
        ],
        out_shape=[
            jax.ShapeDtypeStruct((npad, d_out), jnp.float32),
            jax.ShapeDtypeStruct((SPAD, d_hid), jnp.float32),
        ],
    )(x, brow, phi_w1, b1, phi_w2, b2, rho_w1)

    r = pl.pallas_call(
        _krho,
        grid=(1,),
        in_specs=[
            pl.BlockSpec((SPAD, d_hid), lambda i: (0, 0)),
            pl.BlockSpec((1, d_hid), full),
            pl.BlockSpec((d_hid, DPAD), full),
            pl.BlockSpec((1, DPAD), full),
        ],
        out_specs=pl.BlockSpec((SPAD, DPAD), lambda i: (0, 0)),
        out_shape=jax.ShapeDtypeStruct((SPAD, DPAD), jnp.float32),
    )(s, rb1, rw2p, rb2p)

    mesh = plsc.VectorSubcoreMesh(
        core_axis_name="c", subcore_axis_name="s",
        num_cores=NC, num_subcores=NS)

    out = pl.kernel(
        functools.partial(_gadd_body, nchunk, tail, d_out, cpw),
        out_type=jax.ShapeDtypeStruct((n, d_out), jnp.float32),
        mesh=mesh,
        scratch_types=[
            pltpu.VMEM((cpw, CH), jnp.int32),
            pltpu.VMEM((CH, d_out), jnp.float32),
            pltpu.VMEM((CH, d_out), jnp.float32),
            pltpu.VMEM((CH, DPAD), jnp.float32),
            pltpu.VMEM((CH, DPAD), jnp.float32),
            pltpu.SemaphoreType.DMA,
            pltpu.SemaphoreType.DMA,
            pltpu.SemaphoreType.DMA,
            pltpu.SemaphoreType.DMA,
            pltpu.SemaphoreType.DMA,
            pltpu.SemaphoreType.DMA,
        ],
    )(out1, r, b3d)
    return out


# speedup vs baseline: 1.5716x; 1.2895x over previous
import jax
import jax.numpy as jnp
from jax.experimental import pallas as pl
from jax.experimental.pallas import tpu as pltpu

NSEG = 1000


def _pick_block(n, pref):
    for b in range(pref, 0, -1):
        if n % b == 0 and b % 8 == 0:
            return b
    return n


def _k1(x_ref, brow_ref, w1_ref, b1_ref, w2_ref, b2_ref, rw1_ref,
        out1_ref, s_ref):
    i = pl.program_id(0)
    xb = x_ref[...]
    h = jnp.maximum(
        jnp.dot(xb, w1_ref[...], preferred_element_type=jnp.float32)
        + b1_ref[...], 0.0)
    out1_ref[...] = (
        jnp.dot(h, w2_ref[...], preferred_element_type=jnp.float32)
        + b2_ref[...]).astype(jnp.bfloat16)
    y = jnp.dot(xb, rw1_ref[...], preferred_element_type=jnp.float32)
    b = brow_ref[0]
    nb = b.shape[1]
    seg = jax.lax.broadcasted_iota(jnp.int32, (NSEG, nb), 0).astype(jnp.float32)
    oh_t = (jnp.broadcast_to(b, (NSEG, nb)) == seg).astype(jnp.float32)
    part = jnp.dot(oh_t, y, preferred_element_type=jnp.float32)

    @pl.when(i == 0)
    def _():
        s_ref[...] = part

    @pl.when(i > 0)
    def _():
        s_ref[...] += part


def _krho(s_ref, rb1_ref, rw2_ref, rb2_ref, r_ref):
    r_ref[...] = (
        jnp.dot(jnp.maximum(s_ref[...] + rb1_ref[...], 0.0), rw2_ref[...],
                preferred_element_type=jnp.float32)
        + rb2_ref[...])


def _k2(out1_ref, bcol_ref, r_ref, out_ref):
    bc = bcol_ref[0]
    nb = bc.shape[0]
    seg = jax.lax.broadcasted_iota(jnp.int32, (nb, NSEG), 1).astype(jnp.float32)
    oh = (jnp.broadcast_to(bc, (nb, NSEG)) == seg).astype(jnp.float32)
    out_ref[...] = out1_ref[...].astype(jnp.float32) + jnp.dot(
        oh, r_ref[...], preferred_element_type=jnp.float32)


def kernel(x, batch, phi_w1, phi_b1, phi_w2, phi_b2,
           rho_w1, rho_b1, rho_w2, rho_b2):
    n, d_in = x.shape
    d_hid = phi_w1.shape[1]
    d_out = phi_w2.shape[1]
    bsz = _pick_block(n, 1000)
    nblk = n // bsz
    bsz2 = _pick_block(n, 2000)
    nblk2 = n // bsz2

    bf = batch.astype(jnp.float32)
    brow = bf.reshape(nblk, 1, bsz)
    bcol = bf.reshape(nblk2, bsz2, 1)
    b1 = phi_b1.reshape(1, d_hid)
    b2 = phi_b2.reshape(1, d_out)
    rb1 = rho_b1.reshape(1, d_hid)
    rb2 = rho_b2.reshape(1, d_out)

    full = lambda i: (0, 0)
    out1, s = pl.pallas_call(
        _k1,
        grid=(nblk,),
        in_specs=[
            pl.BlockSpec((bsz, d_in), lambda i: (i, 0)),
            pl.BlockSpec((1, 1, bsz), lambda i: (i, 0, 0)),
            pl.BlockSpec((d_in, d_hid), full),
            pl.BlockSpec((1, d_hid), full),
            pl.BlockSpec((d_hid, d_out), full),
            pl.BlockSpec((1, d_out), full),
            pl.BlockSpec((d_in, d_hid), full),
        ],
        out_specs=[
            pl.BlockSpec((bsz, d_out), lambda i: (i, 0)),
            pl.BlockSpec((NSEG, d_hid), full),
        ],
        out_shape=[
            jax.ShapeDtypeStruct((n, d_out), jnp.bfloat16),
            jax.ShapeDtypeStruct((NSEG, d_hid), jnp.float32),
        ],
    )(x, brow, phi_w1, b1, phi_w2, b2, rho_w1)

    r = pl.pallas_call(
        _krho,
        grid=(1,),
        in_specs=[
            pl.BlockSpec((NSEG, d_hid), lambda i: (0, 0)),
            pl.BlockSpec((1, d_hid), full),
            pl.BlockSpec((d_hid, d_out), full),
            pl.BlockSpec((1, d_out), full),
        ],
        out_specs=pl.BlockSpec((NSEG, d_out), lambda i: (0, 0)),
        out_shape=jax.ShapeDtypeStruct((NSEG, d_out), jnp.float32),
    )(s, rb1, rho_w2, rb2)

    out = pl.pallas_call(
        _k2,
        grid=(nblk2,),
        in_specs=[
            pl.BlockSpec((bsz2, d_out), lambda i: (i, 0)),
            pl.BlockSpec((1, bsz2, 1), lambda i: (i, 0, 0)),
            pl.BlockSpec((NSEG, d_out), full),
        ],
        out_specs=pl.BlockSpec((bsz2, d_out), lambda i: (i, 0)),
        out_shape=jax.ShapeDtypeStruct((n, d_out), jnp.float32),
    )(out1, bcol, r)
    return out


# speedup vs baseline: 1.7848x; 1.1357x over previous
import jax
import jax.numpy as jnp
from jax.experimental import pallas as pl
from jax.experimental.pallas import tpu as pltpu

NSEG = 1000


def _pick_block(n, pref):
    for b in range(pref, 0, -1):
        if n % b == 0 and b % 8 == 0:
            return b
    return n


def _k1(x_ref, brow_ref, w1_ref, b1_ref, w2_ref, b2_ref, rw1_ref,
        out1_ref, s_ref):
    i = pl.program_id(0)
    xb = x_ref[...]
    h = jnp.maximum(
        jnp.dot(xb, w1_ref[...], preferred_element_type=jnp.float32)
        + b1_ref[...], 0.0)
    out1_ref[...] = (
        jnp.dot(h, w2_ref[...], preferred_element_type=jnp.float32)
        + b2_ref[...]).astype(jnp.bfloat16)
    y = jnp.dot(xb, rw1_ref[...], preferred_element_type=jnp.float32)
    b = brow_ref[0]
    nb = b.shape[1]
    seg = jax.lax.broadcasted_iota(jnp.int32, (NSEG, nb), 0).astype(jnp.float32)
    oh_t = (jnp.broadcast_to(b, (NSEG, nb)) == seg).astype(jnp.float32)
    part = jnp.dot(oh_t, y, preferred_element_type=jnp.float32)

    @pl.when(i == 0)
    def _():
        s_ref[...] = part

    @pl.when(i > 0)
    def _():
        s_ref[...] += part


def _krho(s_ref, rb1_ref, rw2_ref, rb2_ref, r_ref):
    r_ref[...] = (
        jnp.dot(jnp.maximum(s_ref[...] + rb1_ref[...], 0.0), rw2_ref[...],
                preferred_element_type=jnp.float32)
        + rb2_ref[...])


def _k2(out1_ref, brow_ref, r_ref, out_ref):
    b = brow_ref[0]
    nb = b.shape[1]
    seg = jax.lax.broadcasted_iota(jnp.int32, (NSEG, nb), 0).astype(jnp.float32)
    oh_t = (jnp.broadcast_to(b, (NSEG, nb)) == seg).astype(jnp.float32)
    add = jax.lax.dot_general(
        oh_t, r_ref[...], (((0,), (0,)), ((), ())),
        preferred_element_type=jnp.float32)
    out_ref[...] = out1_ref[...].astype(jnp.float32) + add


def kernel(x, batch, phi_w1, phi_b1, phi_w2, phi_b2,
           rho_w1, rho_b1, rho_w2, rho_b2):
    n, d_in = x.shape
    d_hid = phi_w1.shape[1]
    d_out = phi_w2.shape[1]
    bsz = _pick_block(n, 1000)
    nblk = n // bsz
    bsz2 = _pick_block(n, 2000)
    nblk2 = n // bsz2

    bf = batch.astype(jnp.float32)
    brow = bf.reshape(nblk, 1, bsz)
    brow2 = bf.reshape(nblk2, 1, bsz2)
    b1 = phi_b1.reshape(1, d_hid)
    b2 = phi_b2.reshape(1, d_out)
    rb1 = rho_b1.reshape(1, d_hid)
    rb2 = rho_b2.reshape(1, d_out)

    full = lambda i: (0, 0)
    out1, s = pl.pallas_call(
        _k1,
        grid=(nblk,),
        in_specs=[
            pl.BlockSpec((bsz, d_in), lambda i: (i, 0)),
            pl.BlockSpec((1, 1, bsz), lambda i: (i, 0, 0)),
            pl.BlockSpec((d_in, d_hid), full),
            pl.BlockSpec((1, d_hid), full),
            pl.BlockSpec((d_hid, d_out), full),
            pl.BlockSpec((1, d_out), full),
            pl.BlockSpec((d_in, d_hid), full),
        ],
        out_specs=[
            pl.BlockSpec((bsz, d_out), lambda i: (i, 0)),
            pl.BlockSpec((NSEG, d_hid), full),
        ],
        out_shape=[
            jax.ShapeDtypeStruct((n, d_out), jnp.bfloat16),
            jax.ShapeDtypeStruct((NSEG, d_hid), jnp.float32),
        ],
    )(x, brow, phi_w1, b1, phi_w2, b2, rho_w1)

    r = pl.pallas_call(
        _krho,
        grid=(1,),
        in_specs=[
            pl.BlockSpec((NSEG, d_hid), lambda i: (0, 0)),
            pl.BlockSpec((1, d_hid), full),
            pl.BlockSpec((d_hid, d_out), full),
            pl.BlockSpec((1, d_out), full),
        ],
        out_specs=pl.BlockSpec((NSEG, d_out), lambda i: (0, 0)),
        out_shape=jax.ShapeDtypeStruct((NSEG, d_out), jnp.float32),
    )(s, rb1, rho_w2, rb2)

    out = pl.pallas_call(
        _k2,
        grid=(nblk2,),
        in_specs=[
            pl.BlockSpec((bsz2, d_out), lambda i: (i, 0)),
            pl.BlockSpec((1, 1, bsz2), lambda i: (i, 0, 0)),
            pl.BlockSpec((NSEG, d_out), full),
        ],
        out_specs=pl.BlockSpec((bsz2, d_out), lambda i: (i, 0)),
        out_shape=jax.ShapeDtypeStruct((n, d_out), jnp.float32),
    )(out1, brow2, r)
    return out


# speedup vs baseline: 2.3163x; 1.2978x over previous
import functools

import jax
import jax.numpy as jnp
from jax.experimental import pallas as pl
from jax.experimental.pallas import tpu as pltpu

NSEG = 1000


def _k1(nrows, x_ref, brow_ref, w1_ref, b1_ref, w2_ref, b2_ref, rw1_ref,
        out1_ref, s_ref):
    i = pl.program_id(0)
    xb = x_ref[...]
    nb = xb.shape[0]
    h = jnp.maximum(
        jnp.dot(xb, w1_ref[...], preferred_element_type=jnp.float32)
        + b1_ref[...], 0.0)
    out1_ref[...] = (
        jax.lax.dot_general(
            w2_ref[...], h, (((0,), (1,)), ((), ())),
            preferred_element_type=jnp.float32)
        + b2_ref[...]).astype(jnp.bfloat16)
    y = jnp.dot(xb, rw1_ref[...], preferred_element_type=jnp.float32)
    rid = jax.lax.broadcasted_iota(jnp.int32, y.shape, 0) + i * nb
    y = jnp.where(rid < nrows, y, 0.0)
    b = brow_ref[0]
    seg = jax.lax.broadcasted_iota(jnp.int32, (NSEG, nb), 0).astype(jnp.float32)
    oh_t = (jnp.broadcast_to(b, (NSEG, nb)) == seg).astype(jnp.float32)
    part = jnp.dot(oh_t, y, preferred_element_type=jnp.float32)

    @pl.when(i == 0)
    def _():
        s_ref[...] = part

    @pl.when(i > 0)
    def _():
        s_ref[...] += part


def _krho(s_ref, rb1_ref, rw2_ref, rb2_ref, r_ref):
    r_ref[...] = (
        jnp.dot(jnp.maximum(s_ref[...] + rb1_ref[...], 0.0), rw2_ref[...],
                preferred_element_type=jnp.float32)
        + rb2_ref[...])


def _k2(out1_ref, brow_ref, r_ref, out_ref):
    b = brow_ref[0]
    nb = b.shape[1]
    seg = jax.lax.broadcasted_iota(jnp.int32, (NSEG, nb), 0).astype(jnp.float32)
    oh_t = (jnp.broadcast_to(b, (NSEG, nb)) == seg).astype(jnp.float32)
    add_t = jax.lax.dot_general(
        r_ref[...], oh_t, (((0,), (0,)), ((), ())),
        preferred_element_type=jnp.float32)
    out_ref[...] = out1_ref[...].astype(jnp.float32) + add_t


def kernel(x, batch, phi_w1, phi_b1, phi_w2, phi_b2,
           rho_w1, rho_b1, rho_w2, rho_b2):
    n, d_in = x.shape
    d_hid = phi_w1.shape[1]
    d_out = phi_w2.shape[1]
    bsz = 1024
    nblk = (n + bsz - 1) // bsz
    bsz2 = 2048
    nblk2 = (n + bsz2 - 1) // bsz2

    bf = batch.astype(jnp.float32)
    bfp = jnp.concatenate(
        [bf, jnp.full((nblk * bsz - n,), float(NSEG), jnp.float32)])
    brow = bfp.reshape(nblk, 1, bsz)
    bfp2 = jnp.concatenate(
        [bf, jnp.full((nblk2 * bsz2 - n,), float(NSEG), jnp.float32)])
    brow2 = bfp2.reshape(nblk2, 1, bsz2)
    b1 = phi_b1.reshape(1, d_hid)
    b2 = phi_b2.reshape(d_out, 1)
    rb1 = rho_b1.reshape(1, d_hid)
    rb2 = rho_b2.reshape(1, d_out)

    full = lambda i: (0, 0)
    out1, s = pl.pallas_call(
        functools.partial(_k1, n),
        grid=(nblk,),
        in_specs=[
            pl.BlockSpec((bsz, d_in), lambda i: (i, 0)),
            pl.BlockSpec((1, 1, bsz), lambda i: (i, 0, 0)),
            pl.BlockSpec((d_in, d_hid), full),
            pl.BlockSpec((1, d_hid), full),
            pl.BlockSpec((d_hid, d_out), full),
            pl.BlockSpec((d_out, 1), full),
            pl.BlockSpec((d_in, d_hid), full),
        ],
        out_specs=[
            pl.BlockSpec((d_out, bsz), lambda i: (0, i)),
            pl.BlockSpec((NSEG, d_hid), full),
        ],
        out_shape=[
            jax.ShapeDtypeStruct((d_out, n), jnp.bfloat16),
            jax.ShapeDtypeStruct((NSEG, d_hid), jnp.float32),
        ],
    )(x, brow, phi_w1, b1, phi_w2, b2, rho_w1)

    r = pl.pallas_call(
        _krho,
        grid=(1,),
        in_specs=[
            pl.BlockSpec((NSEG, d_hid), lambda i: (0, 0)),
            pl.BlockSpec((1, d_hid), full),
            pl.BlockSpec((d_hid, d_out), full),
            pl.BlockSpec((1, d_out), full),
        ],
        out_specs=pl.BlockSpec((NSEG, d_out), lambda i: (0, 0)),
        out_shape=jax.ShapeDtypeStruct((NSEG, d_out), jnp.float32),
    )(s, rb1, rho_w2, rb2)

    out = pl.pallas_call(
        _k2,
        grid=(nblk2,),
        in_specs=[
            pl.BlockSpec((d_out, bsz2), lambda i: (0, i)),
            pl.BlockSpec((1, 1, bsz2), lambda i: (i, 0, 0)),
            pl.BlockSpec((NSEG, d_out), full),
        ],
        out_specs=pl.BlockSpec((d_out, bsz2), lambda i: (0, i)),
        out_shape=jax.ShapeDtypeStruct((d_out, n), jnp.float32),
    )(out1, brow2, r)
    return jnp.transpose(out)


# speedup vs baseline: 2.7543x; 1.1891x over previous
import functools

import jax
import jax.numpy as jnp
from jax.experimental import pallas as pl
from jax.experimental.pallas import tpu as pltpu

NSEG = 1000
SPAD = 1024
WSEG = 128


def _k1(nrows, sref, x_ref, brow_ref, w1_ref, b1_ref, w2_ref, b2_ref,
        rw1_ref, out1_ref, s_ref):
    i = pl.program_id(0)
    xb = x_ref[...]
    nb = xb.shape[0]
    h = jnp.maximum(
        jnp.dot(xb, w1_ref[...], preferred_element_type=jnp.float32)
        + b1_ref[...], 0.0)
    out1_ref[...] = (
        jax.lax.dot_general(
            w2_ref[...], h, (((0,), (1,)), ((), ())),
            preferred_element_type=jnp.float32)
        + b2_ref[...]).astype(jnp.bfloat16)
    y = jnp.dot(xb, rw1_ref[...], preferred_element_type=jnp.float32)
    rid = jax.lax.broadcasted_iota(jnp.int32, y.shape, 0) + i * nb
    y = jnp.where(rid < nrows, y, 0.0)
    b = brow_ref[0]

    @pl.when(i == 0)
    def _():
        s_ref[...] = jnp.zeros_like(s_ref)

    base = pl.multiple_of(sref[i, 0], 8)
    end = sref[i, 1]
    narrow = end - base < WSEG

    @pl.when(narrow)
    def _():
        segw = (jax.lax.broadcasted_iota(jnp.int32, (WSEG, nb), 0)
                + base).astype(jnp.float32)
        ohw = (jnp.broadcast_to(b, (WSEG, nb)) == segw).astype(jnp.float32)
        partw = jnp.dot(ohw, y, preferred_element_type=jnp.float32)
        s_ref[pl.ds(base, WSEG), :] += partw

    @pl.when(jnp.logical_not(narrow))
    def _():
        seg = jax.lax.broadcasted_iota(
            jnp.int32, (SPAD, nb), 0).astype(jnp.float32)
        oh_t = (jnp.broadcast_to(b, (SPAD, nb)) == seg).astype(jnp.float32)
        s_ref[...] += jnp.dot(oh_t, y, preferred_element_type=jnp.float32)


def _krho(s_ref, rb1_ref, rw2_ref, rb2_ref, r_ref):
    r_ref[...] = (
        jnp.dot(jnp.maximum(s_ref[...] + rb1_ref[...], 0.0), rw2_ref[...],
                preferred_element_type=jnp.float32)
        + rb2_ref[...])


def _k2(out1_ref, brow_ref, r_ref, out_ref):
    b = brow_ref[0]
    nb = b.shape[1]
    seg = jax.lax.broadcasted_iota(jnp.int32, (SPAD, nb), 0).astype(jnp.float32)
    oh_t = (jnp.broadcast_to(b, (SPAD, nb)) == seg).astype(jnp.float32)
    add_t = jax.lax.dot_general(
        r_ref[...], oh_t, (((0,), (0,)), ((), ())),
        preferred_element_type=jnp.float32)
    out_ref[...] = out1_ref[...].astype(jnp.float32) + add_t


def kernel(x, batch, phi_w1, phi_b1, phi_w2, phi_b2,
           rho_w1, rho_b1, rho_w2, rho_b2):
    n, d_in = x.shape
    d_hid = phi_w1.shape[1]
    d_out = phi_w2.shape[1]
    bsz = 1024
    nblk = (n + bsz - 1) // bsz
    bsz2 = 2048
    nblk2 = (n + bsz2 - 1) // bsz2

    bf = batch.astype(jnp.float32)
    bfp = jnp.concatenate(
        [bf, jnp.full((nblk * bsz - n,), float(NSEG), jnp.float32)])
    brow = bfp.reshape(nblk, 1, bsz)
    bfp2 = jnp.concatenate(
        [bf, jnp.full((nblk2 * bsz2 - n,), float(NSEG), jnp.float32)])
    brow2 = bfp2.reshape(nblk2, 1, bsz2)
    b1 = phi_b1.reshape(1, d_hid)
    b2 = phi_b2.reshape(d_out, 1)
    rb1 = rho_b1.reshape(1, d_hid)
    rb2 = rho_b2.reshape(1, d_out)

    bi32 = batch.astype(jnp.int32)
    starts = bi32[:: bsz]
    ends = bi32[jnp.minimum(
        (jnp.arange(nblk, dtype=jnp.int32) + 1) * bsz - 1, n - 1)]
    base_w = jnp.minimum((starts // 8) * 8, SPAD - WSEG)
    sinfo = jnp.stack([base_w, ends], axis=1)

    full = lambda i, sr: (0, 0)
    out1, s = pl.pallas_call(
        functools.partial(_k1, n),
        grid_spec=pltpu.PrefetchScalarGridSpec(
            num_scalar_prefetch=1,
            grid=(nblk,),
            in_specs=[
                pl.BlockSpec((bsz, d_in), lambda i, sr: (i, 0)),
                pl.BlockSpec((1, 1, bsz), lambda i, sr: (i, 0, 0)),
                pl.BlockSpec((d_in, d_hid), full),
                pl.BlockSpec((1, d_hid), full),
                pl.BlockSpec((d_hid, d_out), full),
                pl.BlockSpec((d_out, 1), full),
                pl.BlockSpec((d_in, d_hid), full),
            ],
            out_specs=[
                pl.BlockSpec((d_out, bsz), lambda i, sr: (0, i)),
                pl.BlockSpec((SPAD, d_hid), full),
            ],
        ),
        out_shape=[
            jax.ShapeDtypeStruct((d_out, n), jnp.bfloat16),
            jax.ShapeDtypeStruct((SPAD, d_hid), jnp.float32),
        ],
    )(sinfo, x, brow, phi_w1, b1, phi_w2, b2, rho_w1)

    r = pl.pallas_call(
        _krho,
        grid=(1,),
        in_specs=[
            pl.BlockSpec((SPAD, d_hid), lambda i: (0, 0)),
            pl.BlockSpec((1, d_hid), lambda i: (0, 0)),
            pl.BlockSpec((d_hid, d_out), lambda i: (0, 0)),
            pl.BlockSpec((1, d_out), lambda i: (0, 0)),
        ],
        out_specs=pl.BlockSpec((SPAD, d_out), lambda i: (0, 0)),
        out_shape=jax.ShapeDtypeStruct((SPAD, d_out), jnp.float32),
    )(s, rb1, rho_w2, rb2)

    out = pl.pallas_call(
        _k2,
        grid=(nblk2,),
        in_specs=[
            pl.BlockSpec((d_out, bsz2), lambda i: (0, i)),
            pl.BlockSpec((1, 1, bsz2), lambda i: (i, 0, 0)),
            pl.BlockSpec((SPAD, d_out), lambda i: (0, 0)),
        ],
        out_specs=pl.BlockSpec((d_out, bsz2), lambda i: (0, i)),
        out_shape=jax.ShapeDtypeStruct((d_out, n), jnp.float32),
    )(out1, brow2, r)
    return jnp.transpose(out)


# speedup vs baseline: 2.8379x; 1.0304x over previous
import functools

import jax
import jax.numpy as jnp
from jax.experimental import pallas as pl
from jax.experimental.pallas import tpu as pltpu

NSEG = 1000
SPAD = 1024
WSEG = 128


def _k1(nrows, sref, x_ref, brow_ref, w1_ref, b1_ref, w2_ref, b2_ref,
        rw1_ref, out1_ref, s_ref):
    i = pl.program_id(0)
    xb = x_ref[...]
    nb = xb.shape[0]
    h = jnp.maximum(
        jnp.dot(xb, w1_ref[...], preferred_element_type=jnp.float32)
        + b1_ref[...], 0.0)
    out1_ref[...] = (
        jax.lax.dot_general(
            w2_ref[...], h, (((0,), (1,)), ((), ())),
            preferred_element_type=jnp.float32)
        + b2_ref[...]).astype(jnp.bfloat16)
    y = jnp.dot(xb, rw1_ref[...], preferred_element_type=jnp.float32)
    rid = jax.lax.broadcasted_iota(jnp.int32, y.shape, 0) + i * nb
    y = jnp.where(rid < nrows, y, 0.0)
    b = brow_ref[0]

    @pl.when(i == 0)
    def _():
        s_ref[...] = jnp.zeros_like(s_ref)

    base = pl.multiple_of(sref[i, 0], 8)
    end = sref[i, 1]
    narrow = end - base < WSEG

    @pl.when(narrow)
    def _():
        segw = (jax.lax.broadcasted_iota(jnp.int32, (WSEG, nb), 0)
                + base).astype(jnp.float32)
        ohw = (jnp.broadcast_to(b, (WSEG, nb)) == segw).astype(jnp.float32)
        partw = jnp.dot(ohw, y, preferred_element_type=jnp.float32)
        s_ref[pl.ds(base, WSEG), :] += partw

    @pl.when(jnp.logical_not(narrow))
    def _():
        seg = jax.lax.broadcasted_iota(
            jnp.int32, (SPAD, nb), 0).astype(jnp.float32)
        oh_t = (jnp.broadcast_to(b, (SPAD, nb)) == seg).astype(jnp.float32)
        s_ref[...] += jnp.dot(oh_t, y, preferred_element_type=jnp.float32)


def _krho(s_ref, rb1_ref, rw2_ref, rb2_ref, r_ref):
    r_ref[...] = (
        jnp.dot(jnp.maximum(s_ref[...] + rb1_ref[...], 0.0), rw2_ref[...],
                preferred_element_type=jnp.float32)
        + rb2_ref[...])


def _k2(sref, out1_ref, brow_ref, s_ref, rb1_ref, rw2_ref, rb2_ref,
        out_ref, r_scr):
    i = pl.program_id(0)

    @pl.when(i == 0)
    def _():
        r_scr[...] = (
            jnp.dot(jnp.maximum(s_ref[...] + rb1_ref[...], 0.0),
                    rw2_ref[...], preferred_element_type=jnp.float32)
            + rb2_ref[...])

    b = brow_ref[0]
    nb = b.shape[1]
    base = pl.multiple_of(sref[i, 0], 8)
    end = sref[i, 1]
    narrow = end - base < WSEG

    @pl.when(narrow)
    def _():
        segw = (jax.lax.broadcasted_iota(jnp.int32, (WSEG, nb), 0)
                + base).astype(jnp.float32)
        ohw = (jnp.broadcast_to(b, (WSEG, nb)) == segw).astype(jnp.float32)
        add_t = jax.lax.dot_general(
            r_scr[pl.ds(base, WSEG), :], ohw, (((0,), (0,)), ((), ())),
            preferred_element_type=jnp.float32)
        out_ref[...] = out1_ref[...].astype(jnp.float32) + add_t

    @pl.when(jnp.logical_not(narrow))
    def _():
        seg = jax.lax.broadcasted_iota(
            jnp.int32, (SPAD, nb), 0).astype(jnp.float32)
        oh_t = (jnp.broadcast_to(b, (SPAD, nb)) == seg).astype(jnp.float32)
        add_t = jax.lax.dot_general(
            r_scr[...], oh_t, (((0,), (0,)), ((), ())),
            preferred_element_type=jnp.float32)
        out_ref[...] = out1_ref[...].astype(jnp.float32) + add_t


def kernel(x, batch, phi_w1, phi_b1, phi_w2, phi_b2,
           rho_w1, rho_b1, rho_w2, rho_b2):
    n, d_in = x.shape
    d_hid = phi_w1.shape[1]
    d_out = phi_w2.shape[1]
    bsz = 1024
    nblk = (n + bsz - 1) // bsz
    bsz2 = 2048
    nblk2 = (n + bsz2 - 1) // bsz2

    bf = batch.astype(jnp.float32)
    bfp = jnp.concatenate(
        [bf, jnp.full((nblk * bsz - n,), float(NSEG), jnp.float32)])
    brow = bfp.reshape(nblk, 1, bsz)
    bfp2 = jnp.concatenate(
        [bf, jnp.full((nblk2 * bsz2 - n,), float(NSEG), jnp.float32)])
    brow2 = bfp2.reshape(nblk2, 1, bsz2)
    b1 = phi_b1.reshape(1, d_hid)
    b2 = phi_b2.reshape(d_out, 1)
    rb1 = rho_b1.reshape(1, d_hid)
    rb2 = rho_b2.reshape(1, d_out)

    bi32 = batch.astype(jnp.int32)
    starts = bi32[:: bsz]
    ends = bi32[jnp.minimum(
        (jnp.arange(nblk, dtype=jnp.int32) + 1) * bsz - 1, n - 1)]
    base_w = jnp.minimum((starts // 8) * 8, SPAD - WSEG)
    sinfo = jnp.stack([base_w, ends], axis=1)

    full = lambda i, sr: (0, 0)
    out1, s = pl.pallas_call(
        functools.partial(_k1, n),
        grid_spec=pltpu.PrefetchScalarGridSpec(
            num_scalar_prefetch=1,
            grid=(nblk,),
            in_specs=[
                pl.BlockSpec((bsz, d_in), lambda i, sr: (i, 0)),
                pl.BlockSpec((1, 1, bsz), lambda i, sr: (i, 0, 0)),
                pl.BlockSpec((d_in, d_hid), full),
                pl.BlockSpec((1, d_hid), full),
                pl.BlockSpec((d_hid, d_out), full),
                pl.BlockSpec((d_out, 1), full),
                pl.BlockSpec((d_in, d_hid), full),
            ],
            out_specs=[
                pl.BlockSpec((d_out, bsz), lambda i, sr: (0, i)),
                pl.BlockSpec((SPAD, d_hid), full),
            ],
        ),
        out_shape=[
            jax.ShapeDtypeStruct((d_out, n), jnp.bfloat16),
            jax.ShapeDtypeStruct((SPAD, d_hid), jnp.float32),
        ],
    )(sinfo, x, brow, phi_w1, b1, phi_w2, b2, rho_w1)

    starts2 = bi32[:: bsz2]
    ends2 = bi32[jnp.minimum(
        (jnp.arange(nblk2, dtype=jnp.int32) + 1) * bsz2 - 1, n - 1)]
    base_w2 = jnp.minimum((starts2 // 8) * 8, SPAD - WSEG)
    sinfo2 = jnp.stack([base_w2, ends2], axis=1)

    out = pl.pallas_call(
        _k2,
        grid_spec=pltpu.PrefetchScalarGridSpec(
            num_scalar_prefetch=1,
            grid=(nblk2,),
            in_specs=[
                pl.BlockSpec((d_out, bsz2), lambda i, sr: (0, i)),
                pl.BlockSpec((1, 1, bsz2), lambda i, sr: (i, 0, 0)),
                pl.BlockSpec((SPAD, d_hid), full),
                pl.BlockSpec((1, d_hid), full),
                pl.BlockSpec((d_hid, d_out), full),
                pl.BlockSpec((1, d_out), full),
            ],
            out_specs=pl.BlockSpec((d_out, bsz2), lambda i, sr: (0, i)),
            scratch_shapes=[pltpu.VMEM((SPAD, d_out), jnp.float32)],
        ),
        out_shape=jax.ShapeDtypeStruct((d_out, n), jnp.float32),
    )(sinfo2, out1, brow2, s, rb1, rho_w2, rb2)
    return jnp.transpose(out)


# speedup vs baseline: 3.1413x; 1.1069x over previous
import functools

import jax
import jax.numpy as jnp
from jax.experimental import pallas as pl
from jax.experimental.pallas import tpu as pltpu

NSEG = 1000
SPAD = 1024
WSEG = 128


def _k1(nrows, sref, x_ref, brow_ref, w1_ref, b1_ref, w2_ref, b2_ref,
        rw1_ref, out1_ref, s_ref):
    i = pl.program_id(0)
    xb = x_ref[...]
    nb = xb.shape[0]
    h = jnp.maximum(
        jnp.dot(xb, w1_ref[...], preferred_element_type=jnp.float32)
        + b1_ref[...], 0.0)
    out1_ref[...] = (
        jax.lax.dot_general(
            w2_ref[...], h, (((0,), (1,)), ((), ())),
            preferred_element_type=jnp.float32)
        + b2_ref[...]).astype(jnp.bfloat16)
    y = jnp.dot(xb, rw1_ref[...], preferred_element_type=jnp.float32)
    rid = jax.lax.broadcasted_iota(jnp.int32, y.shape, 0) + i * nb
    y = jnp.where(rid < nrows, y, 0.0)
    b = brow_ref[0]

    @pl.when(i == 0)
    def _():
        s_ref[...] = jnp.zeros_like(s_ref)

    base = pl.multiple_of(sref[i, 0], 8)
    end = sref[i, 1]
    narrow = end - base < WSEG

    @pl.when(narrow)
    def _():
        segw = (jax.lax.broadcasted_iota(jnp.int32, (WSEG, nb), 0)
                + base).astype(jnp.float32)
        ohw = (jnp.broadcast_to(b, (WSEG, nb)) == segw).astype(jnp.float32)
        partw = jnp.dot(ohw, y, preferred_element_type=jnp.float32)
        s_ref[pl.ds(base, WSEG), :] += partw

    @pl.when(jnp.logical_not(narrow))
    def _():
        seg = jax.lax.broadcasted_iota(
            jnp.int32, (SPAD, nb), 0).astype(jnp.float32)
        oh_t = (jnp.broadcast_to(b, (SPAD, nb)) == seg).astype(jnp.float32)
        s_ref[...] += jnp.dot(oh_t, y, preferred_element_type=jnp.float32)


def _krho(s_ref, rb1_ref, rw2_ref, rb2_ref, r_ref):
    r_ref[...] = (
        jnp.dot(jnp.maximum(s_ref[...] + rb1_ref[...], 0.0), rw2_ref[...],
                preferred_element_type=jnp.float32)
        + rb2_ref[...])


def _k2(sref, out1_ref, brow_ref, s_ref, rb1_ref, rw2_ref, rb2_ref,
        out_ref, r_scr):
    i = pl.program_id(0)

    @pl.when(i == 0)
    def _():
        r_scr[...] = (
            jnp.dot(jnp.maximum(s_ref[...] + rb1_ref[...], 0.0),
                    rw2_ref[...], preferred_element_type=jnp.float32)
            + rb2_ref[...])

    b = brow_ref[0]
    nb = b.shape[1]
    base = pl.multiple_of(sref[i, 0], 8)
    end = sref[i, 1]
    narrow = end - base < WSEG

    @pl.when(narrow)
    def _():
        segw = (jax.lax.broadcasted_iota(jnp.int32, (WSEG, nb), 0)
                + base).astype(jnp.float32)
        ohw = (jnp.broadcast_to(b, (WSEG, nb)) == segw).astype(jnp.float32)
        add_t = jax.lax.dot_general(
            r_scr[pl.ds(base, WSEG), :], ohw, (((0,), (0,)), ((), ())),
            preferred_element_type=jnp.float32)
        out_ref[...] = out1_ref[...].astype(jnp.float32) + add_t

    @pl.when(jnp.logical_not(narrow))
    def _():
        seg = jax.lax.broadcasted_iota(
            jnp.int32, (SPAD, nb), 0).astype(jnp.float32)
        oh_t = (jnp.broadcast_to(b, (SPAD, nb)) == seg).astype(jnp.float32)
        add_t = jax.lax.dot_general(
            r_scr[...], oh_t, (((0,), (0,)), ((), ())),
            preferred_element_type=jnp.float32)
        out_ref[...] = out1_ref[...].astype(jnp.float32) + add_t


def kernel(x, batch, phi_w1, phi_b1, phi_w2, phi_b2,
           rho_w1, rho_b1, rho_w2, rho_b2):
    n, d_in = x.shape
    d_hid = phi_w1.shape[1]
    d_out = phi_w2.shape[1]
    bsz = 2048
    nblk = (n + bsz - 1) // bsz
    bsz2 = 2048
    nblk2 = (n + bsz2 - 1) // bsz2

    bf = batch.astype(jnp.float32)
    bfp = jnp.concatenate(
        [bf, jnp.full((nblk * bsz - n,), float(NSEG), jnp.float32)])
    brow = bfp.reshape(nblk, 1, bsz)
    bfp2 = jnp.concatenate(
        [bf, jnp.full((nblk2 * bsz2 - n,), float(NSEG), jnp.float32)])
    brow2 = bfp2.reshape(nblk2, 1, bsz2)
    b1 = phi_b1.reshape(1, d_hid)
    b2 = phi_b2.reshape(d_out, 1)
    rb1 = rho_b1.reshape(1, d_hid)
    rb2 = rho_b2.reshape(1, d_out)

    bi32 = batch.astype(jnp.int32)
    starts = bi32[:: bsz]
    ends = bi32[jnp.minimum(
        (jnp.arange(nblk, dtype=jnp.int32) + 1) * bsz - 1, n - 1)]
    base_w = jnp.minimum((starts // 8) * 8, SPAD - WSEG)
    sinfo = jnp.stack([base_w, ends], axis=1)

    full = lambda i, sr: (0, 0)
    out1, s = pl.pallas_call(
        functools.partial(_k1, n),
        grid_spec=pltpu.PrefetchScalarGridSpec(
            num_scalar_prefetch=1,
            grid=(nblk,),
            in_specs=[
                pl.BlockSpec((bsz, d_in), lambda i, sr: (i, 0)),
                pl.BlockSpec((1, 1, bsz), lambda i, sr: (i, 0, 0)),
                pl.BlockSpec((d_in, d_hid), full),
                pl.BlockSpec((1, d_hid), full),
                pl.BlockSpec((d_hid, d_out), full),
                pl.BlockSpec((d_out, 1), full),
                pl.BlockSpec((d_in, d_hid), full),
            ],
            out_specs=[
                pl.BlockSpec((d_out, bsz), lambda i, sr: (0, i)),
                pl.BlockSpec((SPAD, d_hid), full),
            ],
        ),
        out_shape=[
            jax.ShapeDtypeStruct((d_out, n), jnp.bfloat16),
            jax.ShapeDtypeStruct((SPAD, d_hid), jnp.float32),
        ],
    )(sinfo, x, brow, phi_w1, b1, phi_w2, b2, rho_w1)

    starts2 = bi32[:: bsz2]
    ends2 = bi32[jnp.minimum(
        (jnp.arange(nblk2, dtype=jnp.int32) + 1) * bsz2 - 1, n - 1)]
    base_w2 = jnp.minimum((starts2 // 8) * 8, SPAD - WSEG)
    sinfo2 = jnp.stack([base_w2, ends2], axis=1)

    out = pl.pallas_call(
        _k2,
        grid_spec=pltpu.PrefetchScalarGridSpec(
            num_scalar_prefetch=1,
            grid=(nblk2,),
            in_specs=[
                pl.BlockSpec((d_out, bsz2), lambda i, sr: (0, i)),
                pl.BlockSpec((1, 1, bsz2), lambda i, sr: (i, 0, 0)),
                pl.BlockSpec((SPAD, d_hid), full),
                pl.BlockSpec((1, d_hid), full),
                pl.BlockSpec((d_hid, d_out), full),
                pl.BlockSpec((1, d_out), full),
            ],
            out_specs=pl.BlockSpec((d_out, bsz2), lambda i, sr: (0, i)),
            scratch_shapes=[pltpu.VMEM((SPAD, d_out), jnp.float32)],
        ),
        out_shape=jax.ShapeDtypeStruct((d_out, n), jnp.float32),
    )(sinfo2, out1, brow2, s, rb1, rho_w2, rb2)
    return jnp.transpose(out)


# speedup vs baseline: 3.4725x; 1.1054x over previous
import functools

import jax
import jax.numpy as jnp
from jax.experimental import pallas as pl
from jax.experimental.pallas import tpu as pltpu

NSEG = 1000
SPAD = 1024
WSEG = 128


def _k1(nrows, sref, x_ref, brow_ref, w1_ref, b1_ref, w2_ref, b2_ref,
        rw1_ref, out1_ref, s_ref):
    i = pl.program_id(0)
    xb = x_ref[...]
    nb = xb.shape[0]
    h = jnp.maximum(
        jnp.dot(xb, w1_ref[...], preferred_element_type=jnp.float32)
        + b1_ref[...], 0.0)
    out1_ref[...] = (
        jax.lax.dot_general(
            w2_ref[...], h, (((0,), (1,)), ((), ())),
            preferred_element_type=jnp.float32)
        + b2_ref[...]).astype(jnp.bfloat16)
    y = jnp.dot(xb, rw1_ref[...], preferred_element_type=jnp.float32)
    rid = jax.lax.broadcasted_iota(jnp.int32, y.shape, 0) + i * nb
    y = jnp.where(rid < nrows, y, 0.0)
    b = brow_ref[0]

    @pl.when(i == 0)
    def _():
        s_ref[...] = jnp.zeros_like(s_ref)

    base = pl.multiple_of(sref[i, 0], 8)
    end = sref[i, 1]
    narrow = end - base < WSEG

    @pl.when(narrow)
    def _():
        segw = (jax.lax.broadcasted_iota(jnp.int32, (WSEG, nb), 0)
                + base).astype(jnp.float32)
        ohw = (jnp.broadcast_to(b, (WSEG, nb)) == segw).astype(jnp.float32)
        partw = jnp.dot(ohw, y, preferred_element_type=jnp.float32)
        s_ref[pl.ds(base, WSEG), :] += partw

    @pl.when(jnp.logical_not(narrow))
    def _():
        seg = jax.lax.broadcasted_iota(
            jnp.int32, (SPAD, nb), 0).astype(jnp.float32)
        oh_t = (jnp.broadcast_to(b, (SPAD, nb)) == seg).astype(jnp.float32)
        s_ref[...] += jnp.dot(oh_t, y, preferred_element_type=jnp.float32)


def _krho(s_ref, rb1_ref, rw2_ref, rb2_ref, r_ref):
    r_ref[...] = (
        jnp.dot(jnp.maximum(s_ref[...] + rb1_ref[...], 0.0), rw2_ref[...],
                preferred_element_type=jnp.float32)
        + rb2_ref[...])


def _k2(sref, out1_ref, brow_ref, s_ref, rb1_ref, rw2_ref, rb2_ref,
        out_ref, r_scr):
    i = pl.program_id(0)

    @pl.when(i == 0)
    def _():
        r_scr[...] = (
            jnp.dot(jnp.maximum(s_ref[...] + rb1_ref[...], 0.0),
                    rw2_ref[...], preferred_element_type=jnp.float32)
            + rb2_ref[...])

    b = brow_ref[0]
    nb = b.shape[1]
    base = pl.multiple_of(sref[i, 0], 8)
    end = sref[i, 1]
    narrow = end - base < WSEG

    @pl.when(narrow)
    def _():
        segw = (jax.lax.broadcasted_iota(jnp.int32, (WSEG, nb), 0)
                + base).astype(jnp.float32)
        ohw = (jnp.broadcast_to(b, (WSEG, nb)) == segw).astype(jnp.float32)
        add_t = jax.lax.dot_general(
            r_scr[pl.ds(base, WSEG), :], ohw, (((0,), (0,)), ((), ())),
            preferred_element_type=jnp.float32)
        out_ref[...] = out1_ref[...].astype(jnp.float32) + add_t

    @pl.when(jnp.logical_not(narrow))
    def _():
        seg = jax.lax.broadcasted_iota(
            jnp.int32, (SPAD, nb), 0).astype(jnp.float32)
        oh_t = (jnp.broadcast_to(b, (SPAD, nb)) == seg).astype(jnp.float32)
        add_t = jax.lax.dot_general(
            r_scr[...], oh_t, (((0,), (0,)), ((), ())),
            preferred_element_type=jnp.float32)
        out_ref[...] = out1_ref[...].astype(jnp.float32) + add_t


def kernel(x, batch, phi_w1, phi_b1, phi_w2, phi_b2,
           rho_w1, rho_b1, rho_w2, rho_b2):
    n, d_in = x.shape
    d_hid = phi_w1.shape[1]
    d_out = phi_w2.shape[1]
    bsz = 4096
    nblk = (n + bsz - 1) // bsz
    bsz2 = 4096
    nblk2 = (n + bsz2 - 1) // bsz2

    bf = batch.astype(jnp.float32)
    bfp = jnp.concatenate(
        [bf, jnp.full((nblk * bsz - n,), float(NSEG), jnp.float32)])
    brow = bfp.reshape(nblk, 1, bsz)
    bfp2 = jnp.concatenate(
        [bf, jnp.full((nblk2 * bsz2 - n,), float(NSEG), jnp.float32)])
    brow2 = bfp2.reshape(nblk2, 1, bsz2)
    b1 = phi_b1.reshape(1, d_hid)
    b2 = phi_b2.reshape(d_out, 1)
    rb1 = rho_b1.reshape(1, d_hid)
    rb2 = rho_b2.reshape(1, d_out)

    bi32 = batch.astype(jnp.int32)
    starts = bi32[:: bsz]
    ends = bi32[jnp.minimum(
        (jnp.arange(nblk, dtype=jnp.int32) + 1) * bsz - 1, n - 1)]
    base_w = jnp.minimum((starts // 8) * 8, SPAD - WSEG)
    sinfo = jnp.stack([base_w, ends], axis=1)

    full = lambda i, sr: (0, 0)
    out1, s = pl.pallas_call(
        functools.partial(_k1, n),
        grid_spec=pltpu.PrefetchScalarGridSpec(
            num_scalar_prefetch=1,
            grid=(nblk,),
            in_specs=[
                pl.BlockSpec((bsz, d_in), lambda i, sr: (i, 0)),
                pl.BlockSpec((1, 1, bsz), lambda i, sr: (i, 0, 0)),
                pl.BlockSpec((d_in, d_hid), full),
                pl.BlockSpec((1, d_hid), full),
                pl.BlockSpec((d_hid, d_out), full),
                pl.BlockSpec((d_out, 1), full),
                pl.BlockSpec((d_in, d_hid), full),
            ],
            out_specs=[
                pl.BlockSpec((d_out, bsz), lambda i, sr: (0, i)),
                pl.BlockSpec((SPAD, d_hid), full),
            ],
        ),
        out_shape=[
            jax.ShapeDtypeStruct((d_out, n), jnp.bfloat16),
            jax.ShapeDtypeStruct((SPAD, d_hid), jnp.float32),
        ],
    )(sinfo, x, brow, phi_w1, b1, phi_w2, b2, rho_w1)

    starts2 = bi32[:: bsz2]
    ends2 = bi32[jnp.minimum(
        (jnp.arange(nblk2, dtype=jnp.int32) + 1) * bsz2 - 1, n - 1)]
    base_w2 = jnp.minimum((starts2 // 8) * 8, SPAD - WSEG)
    sinfo2 = jnp.stack([base_w2, ends2], axis=1)

    out = pl.pallas_call(
        _k2,
        grid_spec=pltpu.PrefetchScalarGridSpec(
            num_scalar_prefetch=1,
            grid=(nblk2,),
            in_specs=[
                pl.BlockSpec((d_out, bsz2), lambda i, sr: (0, i)),
                pl.BlockSpec((1, 1, bsz2), lambda i, sr: (i, 0, 0)),
                pl.BlockSpec((SPAD, d_hid), full),
                pl.BlockSpec((1, d_hid), full),
                pl.BlockSpec((d_hid, d_out), full),
                pl.BlockSpec((1, d_out), full),
            ],
            out_specs=pl.BlockSpec((d_out, bsz2), lambda i, sr: (0, i)),
            scratch_shapes=[pltpu.VMEM((SPAD, d_out), jnp.float32)],
        ),
        out_shape=jax.ShapeDtypeStruct((d_out, n), jnp.float32),
    )(sinfo2, out1, brow2, s, rb1, rho_w2, rb2)
    return jnp.transpose(out)


# speedup vs baseline: 3.4874x; 1.0043x over previous
import functools

import jax
import jax.numpy as jnp
from jax.experimental import pallas as pl
from jax.experimental.pallas import tpu as pltpu

NSEG = 1000
SPAD = 1024
WSEG = 128


def _k1(nrows, sref, x_ref, brow_ref, w1_ref, b1_ref, w2_ref, b2_ref,
        rw1_ref, out1_ref, s_ref):
    i = pl.program_id(0)
    xb = x_ref[...]
    nb = xb.shape[0]
    h = jnp.maximum(
        jnp.dot(xb, w1_ref[...], preferred_element_type=jnp.float32)
        + b1_ref[...], 0.0)
    out1_ref[...] = (
        jax.lax.dot_general(
            w2_ref[...], h, (((0,), (1,)), ((), ())),
            preferred_element_type=jnp.float32)
        + b2_ref[...]).astype(jnp.bfloat16)
    y = jnp.dot(xb, rw1_ref[...], preferred_element_type=jnp.float32)
    rid = jax.lax.broadcasted_iota(jnp.int32, y.shape, 0) + i * nb
    y = jnp.where(rid < nrows, y, 0.0)
    b = brow_ref[0]

    @pl.when(i == 0)
    def _():
        s_ref[...] = jnp.zeros_like(s_ref)

    base = pl.multiple_of(sref[i, 0], 8)
    end = sref[i, 1]
    narrow = end - base < WSEG

    @pl.when(narrow)
    def _():
        segw = (jax.lax.broadcasted_iota(jnp.int32, (WSEG, nb), 0)
                + base).astype(jnp.float32)
        ohw = (jnp.broadcast_to(b, (WSEG, nb)) == segw).astype(jnp.float32)
        partw = jnp.dot(ohw, y, preferred_element_type=jnp.float32)
        s_ref[pl.ds(base, WSEG), :] += partw

    @pl.when(jnp.logical_not(narrow))
    def _():
        seg = jax.lax.broadcasted_iota(
            jnp.int32, (SPAD, nb), 0).astype(jnp.float32)
        oh_t = (jnp.broadcast_to(b, (SPAD, nb)) == seg).astype(jnp.float32)
        s_ref[...] += jnp.dot(oh_t, y, preferred_element_type=jnp.float32)


def _krho(s_ref, rb1_ref, rw2_ref, rb2_ref, r_ref):
    r_ref[...] = (
        jnp.dot(jnp.maximum(s_ref[...] + rb1_ref[...], 0.0), rw2_ref[...],
                preferred_element_type=jnp.float32)
        + rb2_ref[...])


def _k2(sref, out1_ref, brow_ref, s_ref, rb1_ref, rw2_ref, rb2_ref,
        out_ref, r_scr):
    i = pl.program_id(0)

    @pl.when(i == 0)
    def _():
        r_scr[...] = (
            jnp.dot(jnp.maximum(s_ref[...] + rb1_ref[...], 0.0),
                    rw2_ref[...], preferred_element_type=jnp.float32)
            + rb2_ref[...])

    b = brow_ref[0]
    nb = b.shape[1]
    base = pl.multiple_of(sref[i, 0], 8)
    end = sref[i, 1]
    narrow = end - base < WSEG

    @pl.when(narrow)
    def _():
        segw = (jax.lax.broadcasted_iota(jnp.int32, (WSEG, nb), 0)
                + base).astype(jnp.float32)
        ohw = (jnp.broadcast_to(b, (WSEG, nb)) == segw).astype(jnp.float32)
        add_t = jax.lax.dot_general(
            r_scr[pl.ds(base, WSEG), :], ohw, (((0,), (0,)), ((), ())),
            preferred_element_type=jnp.float32)
        out_ref[...] = out1_ref[...].astype(jnp.float32) + add_t

    @pl.when(jnp.logical_not(narrow))
    def _():
        seg = jax.lax.broadcasted_iota(
            jnp.int32, (SPAD, nb), 0).astype(jnp.float32)
        oh_t = (jnp.broadcast_to(b, (SPAD, nb)) == seg).astype(jnp.float32)
        add_t = jax.lax.dot_general(
            r_scr[...], oh_t, (((0,), (0,)), ((), ())),
            preferred_element_type=jnp.float32)
        out_ref[...] = out1_ref[...].astype(jnp.float32) + add_t


def kernel(x, batch, phi_w1, phi_b1, phi_w2, phi_b2,
           rho_w1, rho_b1, rho_w2, rho_b2):
    n, d_in = x.shape
    d_hid = phi_w1.shape[1]
    d_out = phi_w2.shape[1]
    bsz = 4096
    nblk = (n + bsz - 1) // bsz
    bsz2 = 8192
    nblk2 = (n + bsz2 - 1) // bsz2

    bf = batch.astype(jnp.float32)
    bfp = jnp.concatenate(
        [bf, jnp.full((nblk * bsz - n,), float(NSEG), jnp.float32)])
    brow = bfp.reshape(nblk, 1, bsz)
    bfp2 = jnp.concatenate(
        [bf, jnp.full((nblk2 * bsz2 - n,), float(NSEG), jnp.float32)])
    brow2 = bfp2.reshape(nblk2, 1, bsz2)
    b1 = phi_b1.reshape(1, d_hid)
    b2 = phi_b2.reshape(d_out, 1)
    rb1 = rho_b1.reshape(1, d_hid)
    rb2 = rho_b2.reshape(1, d_out)

    bi32 = batch.astype(jnp.int32)
    starts = bi32[:: bsz]
    ends = bi32[jnp.minimum(
        (jnp.arange(nblk, dtype=jnp.int32) + 1) * bsz - 1, n - 1)]
    base_w = jnp.minimum((starts // 8) * 8, SPAD - WSEG)
    sinfo = jnp.stack([base_w, ends], axis=1)

    full = lambda i, sr: (0, 0)
    out1, s = pl.pallas_call(
        functools.partial(_k1, n),
        grid_spec=pltpu.PrefetchScalarGridSpec(
            num_scalar_prefetch=1,
            grid=(nblk,),
            in_specs=[
                pl.BlockSpec((bsz, d_in), lambda i, sr: (i, 0)),
                pl.BlockSpec((1, 1, bsz), lambda i, sr: (i, 0, 0)),
                pl.BlockSpec((d_in, d_hid), full),
                pl.BlockSpec((1, d_hid), full),
                pl.BlockSpec((d_hid, d_out), full),
                pl.BlockSpec((d_out, 1), full),
                pl.BlockSpec((d_in, d_hid), full),
            ],
            out_specs=[
                pl.BlockSpec((d_out, bsz), lambda i, sr: (0, i)),
                pl.BlockSpec((SPAD, d_hid), full),
            ],
        ),
        out_shape=[
            jax.ShapeDtypeStruct((d_out, n), jnp.bfloat16),
            jax.ShapeDtypeStruct((SPAD, d_hid), jnp.float32),
        ],
    )(sinfo, x, brow, phi_w1, b1, phi_w2, b2, rho_w1)

    starts2 = bi32[:: bsz2]
    ends2 = bi32[jnp.minimum(
        (jnp.arange(nblk2, dtype=jnp.int32) + 1) * bsz2 - 1, n - 1)]
    base_w2 = jnp.minimum((starts2 // 8) * 8, SPAD - WSEG)
    sinfo2 = jnp.stack([base_w2, ends2], axis=1)

    out = pl.pallas_call(
        _k2,
        grid_spec=pltpu.PrefetchScalarGridSpec(
            num_scalar_prefetch=1,
            grid=(nblk2,),
            in_specs=[
                pl.BlockSpec((d_out, bsz2), lambda i, sr: (0, i)),
                pl.BlockSpec((1, 1, bsz2), lambda i, sr: (i, 0, 0)),
                pl.BlockSpec((SPAD, d_hid), full),
                pl.BlockSpec((1, d_hid), full),
                pl.BlockSpec((d_hid, d_out), full),
                pl.BlockSpec((1, d_out), full),
            ],
            out_specs=pl.BlockSpec((d_out, bsz2), lambda i, sr: (0, i)),
            scratch_shapes=[pltpu.VMEM((SPAD, d_out), jnp.float32)],
        ),
        out_shape=jax.ShapeDtypeStruct((d_out, n), jnp.float32),
    )(sinfo2, out1, brow2, s, rb1, rho_w2, rb2)
    return jnp.transpose(out)


# speedup vs baseline: 3.4898x; 1.0007x over previous
import functools

import jax
import jax.numpy as jnp
from jax.experimental import pallas as pl
from jax.experimental.pallas import tpu as pltpu

NSEG = 1000
SPAD = 1024
WSEG = 128


def _k1(nrows, sref, x_ref, brow_ref, w1_ref, b1_ref, w2_ref, b2_ref,
        rw1_ref, out1_ref, s_ref):
    i = pl.program_id(0)
    xb = x_ref[...]
    nb = xb.shape[0]
    h = jnp.maximum(
        jnp.dot(xb, w1_ref[...], preferred_element_type=jnp.float32)
        + b1_ref[...], 0.0)
    out1_ref[...] = (
        jax.lax.dot_general(
            w2_ref[...], h, (((0,), (1,)), ((), ())),
            preferred_element_type=jnp.float32)
        + b2_ref[...]).astype(jnp.bfloat16)
    y = jnp.dot(xb, rw1_ref[...], preferred_element_type=jnp.float32)
    rid = jax.lax.broadcasted_iota(jnp.int32, y.shape, 0) + i * nb
    y = jnp.where(rid < nrows, y, 0.0)
    b = brow_ref[0]

    @pl.when(i == 0)
    def _():
        s_ref[...] = jnp.zeros_like(s_ref)

    base = pl.multiple_of(sref[i, 0], 8)
    end = sref[i, 1]
    narrow = end - base < WSEG

    @pl.when(narrow)
    def _():
        segw = (jax.lax.broadcasted_iota(jnp.int32, (WSEG, nb), 0)
                + base).astype(jnp.float32)
        ohw = (jnp.broadcast_to(b, (WSEG, nb)) == segw).astype(jnp.float32)
        partw = jnp.dot(ohw, y, preferred_element_type=jnp.float32)
        s_ref[pl.ds(base, WSEG), :] += partw

    @pl.when(jnp.logical_not(narrow))
    def _():
        seg = jax.lax.broadcasted_iota(
            jnp.int32, (SPAD, nb), 0).astype(jnp.float32)
        oh_t = (jnp.broadcast_to(b, (SPAD, nb)) == seg).astype(jnp.float32)
        s_ref[...] += jnp.dot(oh_t, y, preferred_element_type=jnp.float32)


def _k2(sref, out1_ref, brow_ref, s_ref, rb1_ref, rw2_ref, rb2_ref,
        out_ref, r_scr):
    i = pl.program_id(0)

    @pl.when(i == 0)
    def _():
        r_scr[...] = (
            jnp.dot(jnp.maximum(s_ref[...] + rb1_ref[...], 0.0),
                    rw2_ref[...], preferred_element_type=jnp.float32)
            + rb2_ref[...])

    b = brow_ref[0]
    nb = b.shape[1]
    base = pl.multiple_of(sref[i, 0], 8)
    end = sref[i, 1]
    narrow = end - base < WSEG

    @pl.when(narrow)
    def _():
        segw = (jax.lax.broadcasted_iota(jnp.int32, (WSEG, nb), 0)
                + base).astype(jnp.float32)
        ohw = (jnp.broadcast_to(b, (WSEG, nb)) == segw).astype(jnp.float32)
        add_t = jax.lax.dot_general(
            r_scr[pl.ds(base, WSEG), :], ohw, (((0,), (0,)), ((), ())),
            preferred_element_type=jnp.float32)
        out_ref[...] = out1_ref[...].astype(jnp.float32) + add_t

    @pl.when(jnp.logical_not(narrow))
    def _():
        seg = jax.lax.broadcasted_iota(
            jnp.int32, (SPAD, nb), 0).astype(jnp.float32)
        oh_t = (jnp.broadcast_to(b, (SPAD, nb)) == seg).astype(jnp.float32)
        add_t = jax.lax.dot_general(
            r_scr[...], oh_t, (((0,), (0,)), ((), ())),
            preferred_element_type=jnp.float32)
        out_ref[...] = out1_ref[...].astype(jnp.float32) + add_t


def kernel(x, batch, phi_w1, phi_b1, phi_w2, phi_b2,
           rho_w1, rho_b1, rho_w2, rho_b2):
    n, d_in = x.shape
    d_hid = phi_w1.shape[1]
    d_out = phi_w2.shape[1]
    bsz = 4096
    nblk = (n + bsz - 1) // bsz
    bsz2 = 8192
    nblk2 = (n + bsz2 - 1) // bsz2

    bf = batch.astype(jnp.float32)
    bfp = jnp.concatenate(
        [bf, jnp.full((nblk * bsz - n,), float(NSEG), jnp.float32)])
    brow = bfp.reshape(nblk, 1, bsz)
    bfp2 = jnp.concatenate(
        [bf, jnp.full((nblk2 * bsz2 - n,), float(NSEG), jnp.float32)])
    brow2 = bfp2.reshape(nblk2, 1, bsz2)
    b1 = phi_b1.reshape(1, d_hid)
    b2 = phi_b2.reshape(d_out, 1)
    rb1 = rho_b1.reshape(1, d_hid)
    rb2 = rho_b2.reshape(1, d_out)

    bi32 = batch.astype(jnp.int32)
    starts = bi32[:: bsz]
    ends = bi32[jnp.minimum(
        (jnp.arange(nblk, dtype=jnp.int32) + 1) * bsz - 1, n - 1)]
    base_w = jnp.minimum((starts // 8) * 8, SPAD - WSEG)
    sinfo = jnp.stack([base_w, ends], axis=1)

    full = lambda i, sr: (0, 0)
    out1, s = pl.pallas_call(
        functools.partial(_k1, n),
        grid_spec=pltpu.PrefetchScalarGridSpec(
            num_scalar_prefetch=1,
            grid=(nblk,),
            in_specs=[
                pl.BlockSpec((bsz, d_in), lambda i, sr: (i, 0)),
                pl.BlockSpec((1, 1, bsz), lambda i, sr: (i, 0, 0)),
                pl.BlockSpec((d_in, d_hid), full),
                pl.BlockSpec((1, d_hid), full),
                pl.BlockSpec((d_hid, d_out), full),
                pl.BlockSpec((d_out, 1), full),
                pl.BlockSpec((d_in, d_hid), full),
            ],
            out_specs=[
                pl.BlockSpec((d_out, bsz), lambda i, sr: (0, i)),
                pl.BlockSpec((SPAD, d_hid), full),
            ],
        ),
        out_shape=[
            jax.ShapeDtypeStruct((d_out, n), jnp.bfloat16),
            jax.ShapeDtypeStruct((SPAD, d_hid), jnp.float32),
        ],
    )(sinfo, x, brow, phi_w1, b1, phi_w2, b2, rho_w1)

    starts2 = bi32[:: bsz2]
    ends2 = bi32[jnp.minimum(
        (jnp.arange(nblk2, dtype=jnp.int32) + 1) * bsz2 - 1, n - 1)]
    base_w2 = jnp.minimum((starts2 // 8) * 8, SPAD - WSEG)
    sinfo2 = jnp.stack([base_w2, ends2], axis=1)

    out = pl.pallas_call(
        _k2,
        grid_spec=pltpu.PrefetchScalarGridSpec(
            num_scalar_prefetch=1,
            grid=(nblk2,),
            in_specs=[
                pl.BlockSpec((d_out, bsz2), lambda i, sr: (0, i)),
                pl.BlockSpec((1, 1, bsz2), lambda i, sr: (i, 0, 0)),
                pl.BlockSpec((SPAD, d_hid), full),
                pl.BlockSpec((1, d_hid), full),
                pl.BlockSpec((d_hid, d_out), full),
                pl.BlockSpec((1, d_out), full),
            ],
            out_specs=pl.BlockSpec((d_out, bsz2), lambda i, sr: (0, i)),
            scratch_shapes=[pltpu.VMEM((SPAD, d_out), jnp.float32)],
        ),
        out_shape=jax.ShapeDtypeStruct((d_out, n), jnp.float32),
    )(sinfo2, out1, brow2, s, rb1, rho_w2, rb2)
    return jnp.transpose(out)
